# baseline (speedup 1.0000x reference)
import functools

import jax
import jax.numpy as jnp
from jax import lax
from jax.experimental import pallas as pl
from jax.experimental.pallas import tpu as pltpu
from jax.experimental.pallas import tpu_sc as plsc

VOCAB = 1000000
D = 64
B = 4096 * 200
NC = 2
NS = 16
NW = NC * NS
B_PER_W = B // NW
CHUNK = 128
N_CHUNKS = B_PER_W // CHUNK


def _make_sc_gather():
    mesh = plsc.VectorSubcoreMesh(core_axis_name="c", subcore_axis_name="s")

    @functools.partial(
        pl.kernel,
        mesh=mesh,
        out_type=jax.ShapeDtypeStruct((B, D), jnp.float32),
        scratch_types=[
            pltpu.VMEM((N_CHUNKS, CHUNK), jnp.int32),
            pltpu.VMEM((2, CHUNK, D), jnp.float32),
            pltpu.SemaphoreType.DMA,
        ],
        compiler_params=pltpu.CompilerParams(use_tc_tiling_on_sc=False),
    )
    def sc_gather(idx_hbm, table_hbm, out_hbm, idx_v, rows_v, sem_g):
        wid = lax.axis_index("s") * NC + lax.axis_index("c")
        base = wid * B_PER_W

        pltpu.sync_copy(idx_hbm.at[wid], idx_v)

        pltpu.async_copy(table_hbm.at[idx_v.at[0]], rows_v.at[0], sem_g)

        def body(i, _):
            for b in range(2):
                j = 2 * i + b
                nxt = j + 1

                @pl.when(nxt < N_CHUNKS)
                def _():
                    pltpu.async_copy(
                        table_hbm.at[idx_v.at[nxt]], rows_v.at[(b + 1) % 2], sem_g
                    )

                pltpu.make_async_copy(
                    table_hbm.at[idx_v.at[j]], rows_v.at[b], sem_g
                ).wait()
                pltpu.sync_copy(
                    rows_v.at[b], out_hbm.at[pl.ds(base + j * CHUNK, CHUNK)]
                )
            return ()

        lax.fori_loop(0, N_CHUNKS // 2, body, ())

    return sc_gather


_sc_gather = _make_sc_gather()


@jax.jit
def kernel(token_ids, weight):
    idx = token_ids.reshape(NW, N_CHUNKS, CHUNK).astype(jnp.int32)
    flat = _sc_gather(idx, weight)
    return flat.reshape(token_ids.shape[0], token_ids.shape[1], D)


# speedup vs baseline: 1.0202x; 1.0202x over previous
import functools

import jax
import jax.numpy as jnp
from jax import lax
from jax.experimental import pallas as pl
from jax.experimental.pallas import tpu as pltpu
from jax.experimental.pallas import tpu_sc as plsc

VOCAB = 1000000
D = 64
B = 4096 * 200
NC = 2
NS = 16
NW = NC * NS
B_PER_W = B // NW
CHUNK = 128
N_CHUNKS = B_PER_W // CHUNK


def _make_sc_gather():
    mesh = plsc.VectorSubcoreMesh(core_axis_name="c", subcore_axis_name="s")

    @functools.partial(
        pl.kernel,
        mesh=mesh,
        out_type=jax.ShapeDtypeStruct((B, D), jnp.float32),
        scratch_types=[
            pltpu.VMEM((N_CHUNKS, CHUNK), jnp.int32),
            pltpu.VMEM((4, CHUNK, D), jnp.float32),
            pltpu.SemaphoreType.DMA,
            pltpu.SemaphoreType.DMA,
        ],
        compiler_params=pltpu.CompilerParams(use_tc_tiling_on_sc=False),
    )
    def sc_gather(idx_hbm, table_hbm, out_hbm, idx_v, rows_v, sem_g, sem_o):
        wid = lax.axis_index("s") * NC + lax.axis_index("c")
        base = wid * B_PER_W

        def out_slice(j):
            return out_hbm.at[pl.ds(base + j * CHUNK, CHUNK)]

        pltpu.sync_copy(idx_hbm.at[wid], idx_v)

        pltpu.async_copy(table_hbm.at[idx_v.at[0]], rows_v.at[0], sem_g)
        pltpu.async_copy(table_hbm.at[idx_v.at[1]], rows_v.at[1], sem_g)

        def body(i, _):
            for b in range(4):
                j = 4 * i + b
                bn = (b + 2) % 4

                @pl.when(j >= 2)
                def _():
                    pltpu.make_async_copy(
                        rows_v.at[bn], out_slice(j - 2), sem_o
                    ).wait()

                @pl.when(j + 2 < N_CHUNKS)
                def _():
                    pltpu.async_copy(
                        table_hbm.at[idx_v.at[j + 2]], rows_v.at[bn], sem_g
                    )

                pltpu.make_async_copy(
                    table_hbm.at[idx_v.at[j]], rows_v.at[b], sem_g
                ).wait()
                pltpu.async_copy(rows_v.at[b], out_slice(j), sem_o)
            return ()

        lax.fori_loop(0, N_CHUNKS // 4, body, ())

        pltpu.make_async_copy(
            rows_v.at[(N_CHUNKS - 2) % 4], out_slice(N_CHUNKS - 2), sem_o
        ).wait()
        pltpu.make_async_copy(
            rows_v.at[(N_CHUNKS - 1) % 4], out_slice(N_CHUNKS - 1), sem_o
        ).wait()

    return sc_gather


_sc_gather = _make_sc_gather()


@jax.jit
def kernel(token_ids, weight):
    idx = token_ids.reshape(NW, N_CHUNKS, CHUNK).astype(jnp.int32)
    flat = _sc_gather(idx, weight)
    return flat.reshape(token_ids.shape[0], token_ids.shape[1], D)


# speedup vs baseline: 1.0209x; 1.0007x over previous
import functools

import jax
import jax.numpy as jnp
from jax import lax
from jax.experimental import pallas as pl
from jax.experimental.pallas import tpu as pltpu
from jax.experimental.pallas import tpu_sc as plsc

VOCAB = 1000000
D = 64
B = 4096 * 200
NC = 2
NS = 16
NW = NC * NS
B_PER_W = B // NW
CHUNK = 256
N_CHUNKS = B_PER_W // CHUNK


def _make_sc_gather():
    mesh = plsc.VectorSubcoreMesh(core_axis_name="c", subcore_axis_name="s")

    @functools.partial(
        pl.kernel,
        mesh=mesh,
        out_type=jax.ShapeDtypeStruct((B, D), jnp.float32),
        scratch_types=[
            pltpu.VMEM((N_CHUNKS, CHUNK), jnp.int32),
            pltpu.VMEM((4, CHUNK, D), jnp.float32),
            pltpu.SemaphoreType.DMA,
            pltpu.SemaphoreType.DMA,
        ],
        compiler_params=pltpu.CompilerParams(use_tc_tiling_on_sc=False),
    )
    def sc_gather(idx_hbm, table_hbm, out_hbm, idx_v, rows_v, sem_g, sem_o):
        wid = lax.axis_index("s") * NC + lax.axis_index("c")
        base = wid * B_PER_W

        def out_slice(j):
            return out_hbm.at[pl.ds(base + j * CHUNK, CHUNK)]

        pltpu.sync_copy(idx_hbm.at[wid], idx_v)

        pltpu.async_copy(table_hbm.at[idx_v.at[0]], rows_v.at[0], sem_g)
        pltpu.async_copy(table_hbm.at[idx_v.at[1]], rows_v.at[1], sem_g)

        def body(i, _):
            for b in range(4):
                j = 4 * i + b
                bn = (b + 2) % 4

                @pl.when(j >= 2)
                def _():
                    pltpu.make_async_copy(
                        rows_v.at[bn], out_slice(j - 2), sem_o
                    ).wait()

                @pl.when(j + 2 < N_CHUNKS)
                def _():
                    pltpu.async_copy(
                        table_hbm.at[idx_v.at[j + 2]], rows_v.at[bn], sem_g
                    )

                pltpu.make_async_copy(
                    table_hbm.at[idx_v.at[j]], rows_v.at[b], sem_g
                ).wait()
                pltpu.async_copy(rows_v.at[b], out_slice(j), sem_o)
            return ()

        lax.fori_loop(0, N_CHUNKS // 4, body, ())

        pltpu.make_async_copy(
            rows_v.at[(N_CHUNKS - 2) % 4], out_slice(N_CHUNKS - 2), sem_o
        ).wait()
        pltpu.make_async_copy(
            rows_v.at[(N_CHUNKS - 1) % 4], out_slice(N_CHUNKS - 1), sem_o
        ).wait()

    return sc_gather


_sc_gather = _make_sc_gather()


@jax.jit
def kernel(token_ids, weight):
    idx = token_ids.reshape(NW, N_CHUNKS, CHUNK).astype(jnp.int32)
    flat = _sc_gather(idx, weight)
    return flat.reshape(token_ids.shape[0], token_ids.shape[1], D)


# speedup vs baseline: 1.3511x; 1.3234x over previous
import functools

import jax
import jax.numpy as jnp
from jax import lax
from jax.experimental import layout as jex_layout
from jax.experimental import pallas as pl
from jax.experimental.pallas import tpu as pltpu
from jax.experimental.pallas import tpu_sc as plsc

VOCAB = 1000000
D = 64
S = 4096
T = 200
NC = 2
NS = 16
NW = NC * NS
S_PER_W = S // NW


def _make_sc_gather():
    mesh = plsc.VectorSubcoreMesh(core_axis_name="c", subcore_axis_name="s")

    @functools.partial(
        pl.kernel,
        mesh=mesh,
        out_type=jax.ShapeDtypeStruct((S, T, 2 * D), jnp.float32),
        scratch_types=[
            pltpu.VMEM((S_PER_W, T), jnp.int32),
            pltpu.VMEM((2, T, D), jnp.float32),
            pltpu.SemaphoreType.DMA,
            pltpu.SemaphoreType.DMA,
        ],
        compiler_params=pltpu.CompilerParams(use_tc_tiling_on_sc=False),
    )
    def sc_gather(idx_hbm, table_hbm, out_hbm, idx_v, rows_v, sem_g, sem_o):
        wid = lax.axis_index("s") * NC + lax.axis_index("c")
        s0 = wid * S_PER_W

        pltpu.sync_copy(idx_hbm.at[wid], idx_v)

        pltpu.async_copy(table_hbm.at[idx_v.at[0]], rows_v.at[0], sem_g)

        def body(i, _):
            for b in range(2):
                j = 2 * i + b
                bn = (b + 1) % 2

                @pl.when(j >= 1)
                def _():
                    pltpu.make_async_copy(
                        rows_v.at[bn],
                        out_hbm.at[s0 + j - 1, :, pl.ds(0, D)],
                        sem_o,
                    ).wait()

                @pl.when(j + 1 < S_PER_W)
                def _():
                    pltpu.async_copy(
                        table_hbm.at[idx_v.at[j + 1]], rows_v.at[bn], sem_g
                    )

                pltpu.make_async_copy(
                    table_hbm.at[idx_v.at[j]], rows_v.at[b], sem_g
                ).wait()
                pltpu.async_copy(
                    rows_v.at[b], out_hbm.at[s0 + j, :, pl.ds(0, D)], sem_o
                )
            return ()

        lax.fori_loop(0, S_PER_W // 2, body, ())

        pltpu.make_async_copy(
            rows_v.at[(S_PER_W - 1) % 2],
            out_hbm.at[s0 + S_PER_W - 1, :, pl.ds(0, D)],
            sem_o,
        ).wait()

    return sc_gather


_sc_gather = _make_sc_gather()


@jax.jit
def kernel(token_ids, weight):
    idx = token_ids.reshape(NW, S_PER_W, T).astype(jnp.int32)
    out_wide = _sc_gather(idx, weight)
    return out_wide[:, :, :D]
